# Initial kernel scaffold; baseline (speedup 1.0000x reference)
import jax
import jax.numpy as jnp
from jax.experimental import pallas as pl


def kernel(x, adj, W1, a1, W2, a2, Wout, aout):
    raise NotImplementedError("write your pallas kernel here")


# speedup vs baseline: 1.7041x; 1.7041x over previous
import functools

import jax
import jax.numpy as jnp
from jax import lax
from jax.experimental import pallas as pl
from jax.experimental.pallas import tpu as pltpu
from jax.experimental.pallas import tpu_sc as plsc

N_NODES = 10000
N_EDGES = 160000
NPAD = 10240
EPAD = 163840
NCORES = 2
NSUB = 16
NWORK = NCORES * NSUB
CHUNK = 64
LANES = 16


def _tc_project(x_pad, W3, a2, nh, din, d):
    nb = NPAD // 1024
    B = 1024

    def body(x_ref, w_ref, a_ref, h_ref, s_ref):
        hblk = jnp.dot(x_ref[...], w_ref[0], preferred_element_type=jnp.float32)
        h_ref[0] = hblk
        s_ref[0] = lax.dot_general(a_ref[0], hblk, (((0,), (1,)), ((), ())))

    return pl.pallas_call(
        body,
        grid=(nh, nb),
        in_specs=[
            pl.BlockSpec((B, din), lambda h, i: (i, 0)),
            pl.BlockSpec((1, din, d), lambda h, i: (h, 0, 0)),
            pl.BlockSpec((1, d, 2), lambda h, i: (h, 0, 0)),
        ],
        out_specs=[
            pl.BlockSpec((1, B, d), lambda h, i: (h, i, 0)),
            pl.BlockSpec((1, 2, B), lambda h, i: (h, 0, i)),
        ],
        out_shape=[
            jax.ShapeDtypeStruct((nh, NPAD, d), jnp.float32),
            jax.ShapeDtypeStruct((nh, 2, NPAD), jnp.float32),
        ],
    )(x_pad, W3, a2)


def _make_sc_edge(d):
    aug = d + LANES
    nj = d // LANES
    per_w = EPAD // NWORK
    nchunk = per_w // CHUNK
    rows_per_tile = NPAD // NSUB

    mesh = plsc.VectorSubcoreMesh(
        core_axis_name="c", subcore_axis_name="s",
        num_cores=NCORES, num_subcores=NSUB)

    @functools.partial(
        pl.kernel,
        out_type=jax.ShapeDtypeStruct((NCORES, NPAD, aug), jnp.float32),
        mesh=mesh,
        compiler_params=pltpu.CompilerParams(
            needs_layout_passes=False, use_tc_tiling_on_sc=False),
        scratch_types=[
            pltpu.VMEM((NPAD,), jnp.float32),
            pltpu.VMEM((NPAD,), jnp.float32),
            pltpu.VMEM((CHUNK,), jnp.int32),
            pltpu.VMEM((CHUNK,), jnp.int32),
            pltpu.VMEM((CHUNK, d), jnp.float32),
            pltpu.VMEM((CHUNK, aug), jnp.float32),
            pltpu.VMEM((CHUNK,), jnp.float32),
            pltpu.VMEM_SHARED((NPAD, aug), jnp.float32),
            pltpu.SemaphoreType.DMA,
        ],
    )
    def sc_edge(hh, ssrc, sdst, srcp, dstp, partial,
                ssrc_t, sdst_t, idx_s, idx_d, rows, staged, ee_t, acc, sem):
        c = lax.axis_index("c")
        s = lax.axis_index("s")
        wid = s * NCORES + c

        def zrow(e, _):
            for j in range(aug // LANES):
                staged[e, pl.ds(j * LANES, LANES)] = jnp.zeros((LANES,), jnp.float32)
            return _
        lax.fori_loop(0, CHUNK, zrow, None)
        for r in range(rows_per_tile // CHUNK):
            pltpu.sync_copy(staged, acc.at[pl.ds(s * rows_per_tile + r * CHUNK, CHUNK)])
        plsc.subcore_barrier()

        pltpu.sync_copy(ssrc, ssrc_t)
        pltpu.sync_copy(sdst, sdst_t)

        col_d = jnp.full((LANES,), d, jnp.int32)
        base0 = wid * per_w

        def chunk_body(k, _):
            base = base0 + k * CHUNK
            pltpu.sync_copy(srcp.at[pl.ds(base, CHUNK)], idx_s)
            pltpu.sync_copy(dstp.at[pl.ds(base, CHUNK)], idx_d)
            pltpu.async_copy(hh.at[idx_d], rows, sem).wait()

            for g in range(CHUNK // LANES):
                iv_s = idx_s[pl.ds(g * LANES, LANES)]
                iv_d = idx_d[pl.ds(g * LANES, LANES)]
                logit = (plsc.load_gather(ssrc_t, [iv_s])
                         + plsc.load_gather(sdst_t, [iv_d]))
                lrelu = jnp.where(logit > 0, logit, 0.2 * logit)
                ee = jnp.exp(-lrelu)
                ee_t[pl.ds(g * LANES, LANES)] = ee
                eidx = lax.iota(jnp.int32, LANES) + g * LANES
                plsc.store_scatter(staged, [eidx, col_d], ee)

            def scale(e, _):
                vs = plsc.load_gather(ee_t, [jnp.full((LANES,), e, jnp.int32)])
                for j in range(nj):
                    staged[e, pl.ds(j * LANES, LANES)] = (
                        rows[e, pl.ds(j * LANES, LANES)] * vs)
                return _
            lax.fori_loop(0, CHUNK, scale, None)

            pltpu.sync_copy(staged, acc.at[idx_s], add=True)
            return _

        lax.fori_loop(0, nchunk, chunk_body, None)
        plsc.subcore_barrier()

        pltpu.sync_copy(acc.at[pl.ds(s * rows_per_tile, rows_per_tile)],
                        partial.at[c, pl.ds(s * rows_per_tile, rows_per_tile)])

    return sc_edge


def _tc_finalize(partials, nh, d, last):
    aug = d + LANES
    nb = NPAD // 1024
    B = 1024

    def body(p_ref, o_ref):
        p = p_ref[0, 0] + p_ref[0, 1]
        rs = jnp.sum(p[:, d:aug], axis=1, keepdims=True)
        v = p[:, :d] / (rs + 1e-16)
        if last:
            o_ref[...] = jax.nn.sigmoid(v)
        else:
            o_ref[...] = jnp.where(v > 0, v, jnp.exp(v) - 1.0)

    return pl.pallas_call(
        body,
        grid=(nh, nb),
        in_specs=[pl.BlockSpec((1, 2, B, aug), lambda h, i: (h, 0, i, 0))],
        out_specs=pl.BlockSpec((B, d), lambda h, i: (i, h)),
        out_shape=jax.ShapeDtypeStruct((NPAD, nh * d), jnp.float32),
    )(partials)


def _layer(x_pad, srcp, dstp, W, a, sc_edge, last):
    nh, din, d = W.shape
    a2 = jnp.stack((a[:, 0, :d], a[:, 0, d:]), axis=-1)
    H3, S3 = _tc_project(x_pad, W, a2, nh, din, d)
    parts = [sc_edge(H3[h], S3[h, 0], S3[h, 1], srcp, dstp) for h in range(nh)]
    partials = jnp.stack(parts, axis=0)
    return _tc_finalize(partials, nh, d, last)


def kernel(x, adj, W1, a1, W2, a2, Wout, aout):
    src = adj[0].astype(jnp.int32)
    dst = adj[1].astype(jnp.int32)
    srcp = jnp.concatenate(
        [src, jnp.full((EPAD - N_EDGES,), N_NODES, jnp.int32)])
    dstp = jnp.concatenate([dst, jnp.zeros((EPAD - N_EDGES,), jnp.int32)])
    x_pad = jnp.pad(x, ((0, NPAD - N_NODES), (0, 0)))

    sc128 = _make_sc_edge(128)
    sc64 = _make_sc_edge(64)

    x1 = _layer(x_pad, srcp, dstp, W1, a1, sc128, last=False)
    x2 = _layer(x1, srcp, dstp, W2, a2, sc128, last=False)
    out = _layer(x2, srcp, dstp, Wout, aout, sc64, last=True)
    return out[:N_NODES]


# speedup vs baseline: 3.0587x; 1.7949x over previous
import functools

import jax
import jax.numpy as jnp
from jax import lax
from jax.experimental import pallas as pl
from jax.experimental.pallas import tpu as pltpu
from jax.experimental.pallas import tpu_sc as plsc

N_NODES = 10000
N_EDGES = 160000
NPAD = 10240
EPAD = 163840
NCORES = 2
NSUB = 16
NWORK = NCORES * NSUB
CHUNK = 64
NCHUNK = EPAD // NWORK // CHUNK
LANES = 16


def _tc_project(x_pad, W3, a2, nh, din, d):
    nb = NPAD // 1024
    B = 1024

    def body(x_ref, w_ref, a_ref, h_ref, s_ref):
        hblk = jnp.dot(x_ref[...], w_ref[0], preferred_element_type=jnp.float32)
        h_ref[0] = hblk
        s_ref[0] = lax.dot_general(a_ref[0], hblk, (((0,), (1,)), ((), ())))

    return pl.pallas_call(
        body,
        grid=(nh, nb),
        in_specs=[
            pl.BlockSpec((B, din), lambda h, i: (i, 0)),
            pl.BlockSpec((1, din, d), lambda h, i: (h, 0, 0)),
            pl.BlockSpec((1, d, 2), lambda h, i: (h, 0, 0)),
        ],
        out_specs=[
            pl.BlockSpec((1, B, d), lambda h, i: (h, i, 0)),
            pl.BlockSpec((1, 2, B), lambda h, i: (h, 0, i)),
        ],
        out_shape=[
            jax.ShapeDtypeStruct((nh, NPAD, d), jnp.float32),
            jax.ShapeDtypeStruct((nh, 2, NPAD), jnp.float32),
        ],
    )(x_pad, W3, a2)


def _make_sc_edge(d):
    aug = d + LANES
    rpt = NPAD // NSUB

    mesh = plsc.VectorSubcoreMesh(
        core_axis_name="c", subcore_axis_name="s",
        num_cores=NCORES, num_subcores=NSUB)

    @functools.partial(
        pl.kernel,
        out_type=jax.ShapeDtypeStruct((NCORES, NPAD, aug), jnp.float32),
        mesh=mesh,
        compiler_params=pltpu.CompilerParams(
            needs_layout_passes=False, use_tc_tiling_on_sc=False),
        scratch_types=[
            pltpu.VMEM((CHUNK, aug), jnp.float32),
            pltpu.VMEM((CHUNK, aug), jnp.float32),
            pltpu.VMEM((CHUNK, aug), jnp.float32),
            pltpu.VMEM((CHUNK, aug), jnp.float32),
            pltpu.VMEM((CHUNK,), jnp.int32),
            pltpu.VMEM((CHUNK,), jnp.int32),
            pltpu.VMEM((CHUNK,), jnp.int32),
            pltpu.VMEM((CHUNK,), jnp.int32),
            pltpu.VMEM((CHUNK,), jnp.int32),
            pltpu.VMEM((CHUNK,), jnp.int32),
            pltpu.VMEM((CHUNK,), jnp.int32),
            pltpu.VMEM((CHUNK,), jnp.int32),
            pltpu.VMEM((CHUNK,), jnp.float32),
            pltpu.VMEM((CHUNK,), jnp.float32),
            pltpu.VMEM((CHUNK,), jnp.float32),
            pltpu.VMEM((CHUNK,), jnp.float32),
            pltpu.VMEM((CHUNK,), jnp.float32),
            pltpu.VMEM_SHARED((NPAD, aug), jnp.float32),
            pltpu.SemaphoreType.DMA((2,)),
            pltpu.SemaphoreType.DMA((4,)),
            pltpu.SemaphoreType.DMA((2,)),
        ],
    )
    def sc_edge(haug, ssrc, srcp3, dstp3, partial,
                rows0, rows1, staged0, staged1,
                si0, si1, si2, si3, di0, di1, di2, di3,
                sv0, sv1, sv2, sv3, ee_t, acc, sem_g, sem_i, sem_sc):
        ROWS = (rows0, rows1)
        STAGED = (staged0, staged1)
        SI = (si0, si1, si2, si3)
        DI = (di0, di1, di2, di3)
        SV = (sv0, sv1, sv2, sv3)
        c = lax.axis_index("c")
        s = lax.axis_index("s")
        wid = s * NCORES + c
        col_sd = jnp.full((LANES,), d + 1, jnp.int32)

        def zrow(e, carry):
            for j in range(aug // LANES):
                staged0[e, pl.ds(j * LANES, LANES)] = jnp.zeros(
                    (LANES,), jnp.float32)
            return carry
        lax.fori_loop(0, CHUNK, zrow, None)
        for r in range(rpt // CHUNK):
            pltpu.sync_copy(staged0, acc.at[pl.ds(s * rpt + r * CHUNK, CHUNK)])
        plsc.subcore_barrier()

        for k in (0, 1):
            pltpu.sync_copy(srcp3.at[wid, k], SI[k])
            pltpu.sync_copy(dstp3.at[wid, k], DI[k])
            pltpu.async_copy(haug.at[DI[k]], ROWS[k], sem_g.at[k])
            pltpu.async_copy(ssrc.at[SI[k]], SV[k], sem_g.at[k])

        def half(k, b, bi, first, prefetch):
            bi2 = (bi + 2) % 4
            rows_b, staged_b = ROWS[b], STAGED[b]
            pltpu.make_async_copy(haug.at[DI[bi]], rows_b, sem_g.at[b]).wait()
            pltpu.make_async_copy(ssrc.at[SI[bi]], SV[bi], sem_g.at[b]).wait()
            for g in range(CHUNK // LANES):
                sval = SV[bi][pl.ds(g * LANES, LANES)]
                eidx = lax.iota(jnp.int32, LANES) + g * LANES
                dval = plsc.load_gather(rows_b, [eidx, col_sd])
                logit = sval + dval
                ee = jnp.exp(-jnp.where(logit > 0, logit, 0.2 * logit))
                ee_t[pl.ds(g * LANES, LANES)] = ee
            if not first:
                pltpu.make_async_copy(
                    staged_b, acc.at[SI[bi]], sem_sc.at[b]).wait()
            if prefetch:
                pltpu.async_copy(srcp3.at[wid, k + 2], SI[bi2], sem_i.at[bi2])
                pltpu.async_copy(dstp3.at[wid, k + 2], DI[bi2], sem_i.at[bi2])
            @plsc.parallel_loop(0, CHUNK, unroll=4)
            def scale(e):
                vs = plsc.load_gather(ee_t, [jnp.full((LANES,), e, jnp.int32)])
                for j in range(aug // LANES):
                    staged_b[e, pl.ds(j * LANES, LANES)] = (
                        rows_b[e, pl.ds(j * LANES, LANES)] * vs)
            pltpu.async_copy(staged_b, acc.at[SI[bi]], sem_sc.at[b], add=True)
            if prefetch:
                pltpu.make_async_copy(
                    srcp3.at[wid, k + 2], SI[bi2], sem_i.at[bi2]).wait()
                pltpu.make_async_copy(
                    dstp3.at[wid, k + 2], DI[bi2], sem_i.at[bi2]).wait()
                pltpu.async_copy(haug.at[DI[bi2]], rows_b, sem_g.at[b])
                pltpu.async_copy(ssrc.at[SI[bi2]], SV[bi2], sem_g.at[b])

        half(0, 0, 0, first=True, prefetch=True)
        half(1, 1, 1, first=True, prefetch=True)
        half(2, 0, 2, first=False, prefetch=True)
        half(3, 1, 3, first=False, prefetch=True)

        def group(m, carry):
            k0 = 4 * m
            half(k0, 0, 0, first=False, prefetch=True)
            half(k0 + 1, 1, 1, first=False, prefetch=True)
            half(k0 + 2, 0, 2, first=False, prefetch=True)
            half(k0 + 3, 1, 3, first=False, prefetch=True)
            return carry
        lax.fori_loop(1, NCHUNK // 4 - 1, group, None)

        k0 = NCHUNK - 4
        half(k0, 0, 0, first=False, prefetch=True)
        half(k0 + 1, 1, 1, first=False, prefetch=True)
        half(k0 + 2, 0, 2, first=False, prefetch=False)
        half(k0 + 3, 1, 3, first=False, prefetch=False)

        pltpu.make_async_copy(staged0, acc.at[SI[2]], sem_sc.at[0]).wait()
        pltpu.make_async_copy(staged1, acc.at[SI[3]], sem_sc.at[1]).wait()
        plsc.subcore_barrier()

        pltpu.sync_copy(acc.at[pl.ds(s * rpt, rpt)],
                        partial.at[c, pl.ds(s * rpt, rpt)])

    return sc_edge


def _tc_finalize(partials, nh, d, last):
    aug = d + LANES
    nb = NPAD // 1024
    B = 1024

    def body(p_ref, o_ref):
        p = p_ref[0, 0] + p_ref[0, 1]
        rs = p[:, d:d + 1]
        v = p[:, :d] / (rs + 1e-16)
        if last:
            o_ref[...] = jax.nn.sigmoid(v)
        else:
            o_ref[...] = jnp.where(v > 0, v, jnp.exp(v) - 1.0)

    return pl.pallas_call(
        body,
        grid=(nh, nb),
        in_specs=[pl.BlockSpec((1, 2, B, aug), lambda h, i: (h, 0, i, 0))],
        out_specs=pl.BlockSpec((B, d), lambda h, i: (i, h)),
        out_shape=jax.ShapeDtypeStruct((NPAD, nh * d), jnp.float32),
    )(partials)


def _layer(x_pad, srcp3, dstp3, W, a, sc_edge, last):
    nh, din, d = W.shape
    a2 = jnp.stack((a[:, 0, :d], a[:, 0, d:]), axis=-1)
    H3, S3 = _tc_project(x_pad, W, a2, nh, din, d)
    ones = jnp.ones((nh, NPAD, 1), jnp.float32)
    sdcol = S3[:, 1, :, None]
    zeros = jnp.zeros((nh, NPAD, LANES - 2), jnp.float32)
    haug = jnp.concatenate([H3, ones, sdcol, zeros], axis=2)
    parts = [sc_edge(haug[h], S3[h, 0], srcp3, dstp3) for h in range(nh)]
    partials = jnp.stack(parts, axis=0)
    return _tc_finalize(partials, nh, d, last)


def kernel(x, adj, W1, a1, W2, a2, Wout, aout):
    src = adj[0].astype(jnp.int32)
    dst = adj[1].astype(jnp.int32)
    srcp3 = jnp.concatenate(
        [src, jnp.full((EPAD - N_EDGES,), N_NODES, jnp.int32)]
    ).reshape(NWORK, NCHUNK, CHUNK)
    dstp3 = jnp.concatenate(
        [dst, jnp.zeros((EPAD - N_EDGES,), jnp.int32)]
    ).reshape(NWORK, NCHUNK, CHUNK)
    x_pad = jnp.pad(x, ((0, NPAD - N_NODES), (0, 0)))

    sc128 = _make_sc_edge(128)
    sc64 = _make_sc_edge(64)

    x1 = _layer(x_pad, srcp3, dstp3, W1, a1, sc128, last=False)
    x2 = _layer(x1, srcp3, dstp3, W2, a2, sc128, last=False)
    out = _layer(x2, srcp3, dstp3, Wout, aout, sc64, last=True)
    return out[:N_NODES]
